# baseline (speedup 1.0000x reference)
import jax
import jax.numpy as jnp
from jax import lax
from jax.experimental import pallas as pl
from jax.experimental.pallas import tpu as pltpu
from jax.experimental.pallas import tpu_sc as plsc

NU = 5000
NI = 5000
N = NU + NI
NP = 10240
D = 128
E = 320000
NLAYERS = 3

NC = 2
NS = 16
B = 50
EPT = E // NS
NBATCH = EPT // B
CH = 40
NCHUNK = NBATCH // CH
DEPTH = 4
SG = 2
RPT = NP // NS
ZROWS = 40
DEG_LAG = 8


def _zero_block(ref, nrows):
    def row(i, _):
        def col(j, _):
            ref[i, pl.ds(j * 16, 16)] = jnp.zeros((16,), jnp.float32)
            return 0
        return lax.fori_loop(0, D // 16, col, 0)
    lax.fori_loop(0, nrows, row, 0)


def _deg_body(row_hbm, out_hbm, idx_v, ones_v, dbuf_v, sem, deg_acc):
    c = lax.axis_index("c")
    s = lax.axis_index("s")
    one_offs = list(range(0, B - 15, 16))
    if B % 16:
        one_offs.append(B - 16)
    for o in one_offs:
        ones_v[pl.ds(o, 16)] = jnp.ones((16,), jnp.float32)
    for j in range(RPT // 16):
        dbuf_v[pl.ds(j * 16, 16)] = jnp.zeros((16,), jnp.float32)
    pltpu.sync_copy(dbuf_v, deg_acc.at[pl.ds(s * RPT, RPT)])
    pltpu.sync_copy(row_hbm.at[c, s], idx_v)
    plsc.subcore_barrier()

    def wait_one():
        pltpu.make_async_copy(ones_v, deg_acc.at[idx_v.at[0]], sem).wait()

    def body(b, _):
        pltpu.async_copy(ones_v, deg_acc.at[idx_v.at[b]], sem, add=True)

        @pl.when(b >= DEG_LAG)
        def _():
            wait_one()
        return 0
    lax.fori_loop(0, NBATCH, body, 0)
    for _ in range(DEG_LAG):
        wait_one()
    plsc.subcore_barrier()
    pltpu.sync_copy(deg_acc.at[pl.ds(s * RPT, RPT)], dbuf_v)
    pltpu.sync_copy(dbuf_v, out_hbm.at[c, pl.ds(s * RPT, RPT)])


def _spmm_body(x_hbm, row_hbm, col_hbm, out_hbm,
               idxr0, idxr1, idxc0, idxc1,
               rows0, rows1, rows2, rows3,
               semi, semg0, semg1, semg2, semg3,
               sems0, sems1, sems2, sems3, acc):
    c = lax.axis_index("c")
    s = lax.axis_index("s")
    idxr = (idxr0, idxr1)
    idxc = (idxc0, idxc1)
    rows = (rows0, rows1, rows2, rows3)
    semg = (semg0, semg1, semg2, semg3)
    sems = (sems0, sems1, sems2, sems3)

    _zero_block(rows0, ZROWS)
    r0 = s * RPT
    for k in range(RPT // ZROWS):
        pltpu.sync_copy(rows0.at[pl.ds(0, ZROWS)],
                        acc.at[pl.ds(r0 + k * ZROWS, ZROWS)])
    pltpu.sync_copy(row_hbm.at[c, s, pl.ds(0, CH)], idxr[0])
    pltpu.sync_copy(col_hbm.at[c, s, pl.ds(0, CH)], idxc[0])
    plsc.subcore_barrier()

    for ch in range(NCHUNK):
        ib = ch % 2
        if ch > 0:
            pltpu.make_async_copy(row_hbm.at[c, s, pl.ds(0, CH)],
                                  idxr[ib], semi).wait()
            pltpu.make_async_copy(col_hbm.at[c, s, pl.ds(0, CH)],
                                  idxc[ib], semi).wait()
        if ch < NCHUNK - 1:
            nb = (ch + 1) % 2
            off = (ch + 1) * CH
            pltpu.async_copy(row_hbm.at[c, s, pl.ds(off, CH)], idxr[nb], semi)
            pltpu.async_copy(col_hbm.at[c, s, pl.ds(off, CH)], idxc[nb], semi)
        for k in range(DEPTH):
            pltpu.async_copy(x_hbm.at[idxc[ib].at[k]], rows[k], semg[k])

        def group(g, _, ib=ib):
            for k in range(DEPTH):
                m = g * DEPTH + k
                kj = (k - (SG - 1)) % DEPTH
                j = m - (SG - 1)
                pltpu.make_async_copy(x_hbm.at[idxc[ib].at[0]],
                                      rows[k], semg[k]).wait()
                pltpu.async_copy(rows[k], acc.at[idxr[ib].at[m]],
                                 sems[k], add=True)

                @pl.when(j >= 0)
                def _():
                    pltpu.make_async_copy(rows[kj], acc.at[idxr[ib].at[0]],
                                          sems[kj]).wait()

                @pl.when(jnp.logical_and(j >= 0, j + DEPTH < CH))
                def _():
                    pltpu.async_copy(x_hbm.at[idxc[ib].at[j + DEPTH]],
                                     rows[kj], semg[kj])
            return 0
        lax.fori_loop(0, CH // DEPTH, group, 0)
        for t in range(SG - 1):
            j = CH - (SG - 1) + t
            pltpu.make_async_copy(rows[j % DEPTH], acc.at[idxr[ib].at[0]],
                                  sems[j % DEPTH]).wait()

    plsc.subcore_barrier()
    for k in range(RPT // ZROWS):
        pltpu.sync_copy(acc.at[pl.ds(r0 + k * ZROWS, ZROWS)],
                        rows0.at[pl.ds(0, ZROWS)])
        pltpu.sync_copy(rows0.at[pl.ds(0, ZROWS)],
                        out_hbm.at[c, pl.ds(r0 + k * ZROWS, ZROWS)])


_MESH = None


def _mesh():
    global _MESH
    if _MESH is None:
        _MESH = plsc.VectorSubcoreMesh(core_axis_name="c", subcore_axis_name="s")
    return _MESH


def _sc_deg(row4):
    f = pl.kernel(
        _deg_body,
        out_type=jax.ShapeDtypeStruct((NC, NP), jnp.float32),
        mesh=_mesh(),
        scratch_types=[
            pltpu.VMEM((NBATCH, B), jnp.int32),
            pltpu.VMEM((B,), jnp.float32),
            pltpu.VMEM((RPT,), jnp.float32),
            pltpu.SemaphoreType.DMA,
            pltpu.VMEM_SHARED((NP,), jnp.float32),
        ],
    )
    return f(row4)


def _sc_spmm(x, row4, col4):
    f = pl.kernel(
        _spmm_body,
        out_type=jax.ShapeDtypeStruct((NC, NP, D), jnp.float32),
        mesh=_mesh(),
        scratch_types=[
            pltpu.VMEM((CH, B), jnp.int32),
            pltpu.VMEM((CH, B), jnp.int32),
            pltpu.VMEM((CH, B), jnp.int32),
            pltpu.VMEM((CH, B), jnp.int32),
            pltpu.VMEM((B, D), jnp.float32),
            pltpu.VMEM((B, D), jnp.float32),
            pltpu.VMEM((B, D), jnp.float32),
            pltpu.VMEM((B, D), jnp.float32),
            pltpu.SemaphoreType.DMA,
            pltpu.SemaphoreType.DMA,
            pltpu.SemaphoreType.DMA,
            pltpu.SemaphoreType.DMA,
            pltpu.SemaphoreType.DMA,
            pltpu.SemaphoreType.DMA,
            pltpu.SemaphoreType.DMA,
            pltpu.SemaphoreType.DMA,
            pltpu.SemaphoreType.DMA,
            pltpu.VMEM_SHARED((NP, D), jnp.float32),
        ],
    )
    return f(x, row4, col4)


_R = 2000


def _dis_body(pdeg_ref, dis_ref):
    deg = pdeg_ref[0] + pdeg_ref[1]
    dis_ref[...] = jnp.where(deg > 0, lax.rsqrt(deg), 0.0)


def _tc_dis(pdeg):
    p3 = pdeg.reshape(NC, NP // 128, 128)
    f = pl.pallas_call(
        _dis_body,
        out_shape=jax.ShapeDtypeStruct((NP // 128, 128), jnp.float32),
    )
    return f(p3)


def _prep_body(dis_ref, emb_ref, x_ref):
    i = pl.program_id(0)
    d = dis_ref[pl.ds(i * _R, _R), :]
    x_ref[...] = emb_ref[...] * d


def _tc_prep(dis_col, emb0):
    f = pl.pallas_call(
        _prep_body,
        grid=(N // _R,),
        in_specs=[
            pl.BlockSpec((N, 1), lambda i: (0, 0)),
            pl.BlockSpec((_R, D), lambda i: (i, 0)),
        ],
        out_specs=pl.BlockSpec((_R, D), lambda i: (i, 0)),
        out_shape=jax.ShapeDtypeStruct((N, D), jnp.float32),
    )
    return f(dis_col, emb0)


def _comb_body(p_ref, dis_ref, acc_ref, accout_ref, x_ref):
    i = pl.program_id(0)
    t = p_ref[0] + p_ref[1]
    d = dis_ref[pl.ds(i * _R, _R), :]
    e = d * t
    accout_ref[...] = acc_ref[...] + e
    x_ref[...] = d * e


def _final_body(p_ref, dis_ref, acc_ref, out_ref):
    i = pl.program_id(0)
    t = p_ref[0] + p_ref[1]
    d = dis_ref[pl.ds(i * _R, _R), :]
    out_ref[...] = 0.25 * (acc_ref[...] + d * t)


def _tc_combine(p, dis_col, acc, last):
    body = _final_body if last else _comb_body
    oshape = jax.ShapeDtypeStruct((N, D), jnp.float32)
    ospec = pl.BlockSpec((_R, D), lambda i: (i, 0))
    f = pl.pallas_call(
        body,
        grid=(N // _R,),
        in_specs=[
            pl.BlockSpec((NC, _R, D), lambda i: (0, i, 0)),
            pl.BlockSpec((N, 1), lambda i: (0, 0)),
            pl.BlockSpec((_R, D), lambda i: (i, 0)),
        ],
        out_specs=ospec if last else (ospec, ospec),
        out_shape=oshape if last else (oshape, oshape),
    )
    return f(p, dis_col, acc)


def kernel(edge_index, user_emb, item_emb):
    row = jnp.concatenate([edge_index[0], edge_index[1]])
    col = jnp.concatenate([edge_index[1], edge_index[0]])
    emb0 = jnp.concatenate([user_emb, item_emb], axis=0)
    row4 = row.reshape(NC, NS, NBATCH, B)
    col4 = col.reshape(NC, NS, NBATCH, B)

    pdeg = _sc_deg(row4)
    dis_flat = _tc_dis(pdeg)
    dis_col = dis_flat.reshape(-1)[:N][:, None]

    x = _tc_prep(dis_col, emb0)
    acc = emb0
    for l in range(NLAYERS):
        p = _sc_spmm(x, row4, col4)
        if l < NLAYERS - 1:
            acc, x = _tc_combine(p, dis_col, acc, last=False)
        else:
            final = _tc_combine(p, dis_col, acc, last=True)
    return (final[:NU], final[NU:])


# speedup vs baseline: 1.1061x; 1.1061x over previous
import jax
import jax.numpy as jnp
from jax import lax
from jax.experimental import pallas as pl
from jax.experimental.pallas import tpu as pltpu
from jax.experimental.pallas import tpu_sc as plsc

NU = 5000
NI = 5000
N = NU + NI
NP = 10240
D = 128
E = 320000
NLAYERS = 3

NC = 2
NS = 16
B = 50
EPT = E // NS
NBATCH = EPT // B
CH = 40
NCHUNK = NBATCH // CH
DEPTH = 5
RPT = NP // NS
ZROWS = 40
DEG_LAG = 8


def _zero_block(ref, nrows):
    def row(i, _):
        def col(j, _):
            ref[i, pl.ds(j * 16, 16)] = jnp.zeros((16,), jnp.float32)
            return 0
        return lax.fori_loop(0, D // 16, col, 0)
    lax.fori_loop(0, nrows, row, 0)


def _deg_body(row_hbm, out_hbm, idx_v, ones_v, dbuf_v, sem, deg_acc):
    c = lax.axis_index("c")
    s = lax.axis_index("s")
    one_offs = list(range(0, B - 15, 16))
    if B % 16:
        one_offs.append(B - 16)
    for o in one_offs:
        ones_v[pl.ds(o, 16)] = jnp.ones((16,), jnp.float32)
    for j in range(RPT // 16):
        dbuf_v[pl.ds(j * 16, 16)] = jnp.zeros((16,), jnp.float32)
    pltpu.sync_copy(dbuf_v, deg_acc.at[pl.ds(s * RPT, RPT)])
    pltpu.sync_copy(row_hbm.at[c, s], idx_v)
    plsc.subcore_barrier()

    def wait_one():
        pltpu.make_async_copy(ones_v, deg_acc.at[idx_v.at[0]], sem).wait()

    def body(b, _):
        pltpu.async_copy(ones_v, deg_acc.at[idx_v.at[b]], sem, add=True)

        @pl.when(b >= DEG_LAG)
        def _():
            wait_one()
        return 0
    lax.fori_loop(0, NBATCH, body, 0)
    for _ in range(DEG_LAG):
        wait_one()
    plsc.subcore_barrier()
    pltpu.sync_copy(deg_acc.at[pl.ds(s * RPT, RPT)], dbuf_v)
    pltpu.sync_copy(dbuf_v, out_hbm.at[c, pl.ds(s * RPT, RPT)])


def _spmm_body(x_hbm, comb_hbm, out_hbm,
               idx0, idx1,
               rows0, rows1, rows2, rows3, rows4,
               semi, semg0, semg1, semg2, semg3, semg4,
               sems0, sems1, sems2, sems3, sems4, acc):
    c = lax.axis_index("c")
    s = lax.axis_index("s")
    idx = (idx0, idx1)
    rows = (rows0, rows1, rows2, rows3, rows4)
    semg = (semg0, semg1, semg2, semg3, semg4)
    sems = (sems0, sems1, sems2, sems3, sems4)

    def colv(ib, b):
        return idx[ib].at[b, pl.ds(0, B)]

    def rowv(ib, b):
        return idx[ib].at[b, pl.ds(64, B)]

    _zero_block(rows0, ZROWS)
    r0 = s * RPT
    for k in range(RPT // ZROWS):
        pltpu.sync_copy(rows0.at[pl.ds(0, ZROWS)],
                        acc.at[pl.ds(r0 + k * ZROWS, ZROWS)])
    pltpu.sync_copy(comb_hbm.at[c, s, pl.ds(0, CH)], idx[0])
    plsc.subcore_barrier()

    for ch in range(NCHUNK):
        ib = ch % 2
        if ch > 0:
            pltpu.make_async_copy(comb_hbm.at[c, s, pl.ds(0, CH)],
                                  idx[ib], semi).wait()
        if ch < NCHUNK - 1:
            nb = (ch + 1) % 2
            off = (ch + 1) * CH
            pltpu.async_copy(comb_hbm.at[c, s, pl.ds(off, CH)], idx[nb], semi)
        for k in range(DEPTH):
            pltpu.async_copy(x_hbm.at[colv(ib, k)], rows[k], semg[k])

        def group(g, _, ib=ib):
            for k in range(DEPTH):
                b = g * DEPTH + k
                pltpu.make_async_copy(x_hbm.at[colv(ib, 0)],
                                      rows[k], semg[k]).wait()
                pltpu.async_copy(rows[k], acc.at[rowv(ib, b)],
                                 sems[k], add=True)
                pltpu.make_async_copy(rows[k], acc.at[rowv(ib, 0)],
                                      sems[k]).wait()

                @pl.when(b + DEPTH < CH)
                def _():
                    pltpu.async_copy(x_hbm.at[colv(ib, b + DEPTH)],
                                     rows[k], semg[k])
            return 0
        lax.fori_loop(0, CH // DEPTH, group, 0)

    plsc.subcore_barrier()
    for k in range(RPT // ZROWS):
        pltpu.sync_copy(acc.at[pl.ds(r0 + k * ZROWS, ZROWS)],
                        rows0.at[pl.ds(0, ZROWS)])
        pltpu.sync_copy(rows0.at[pl.ds(0, ZROWS)],
                        out_hbm.at[c, pl.ds(r0 + k * ZROWS, ZROWS)])


_MESH = None


def _mesh():
    global _MESH
    if _MESH is None:
        _MESH = plsc.VectorSubcoreMesh(core_axis_name="c", subcore_axis_name="s")
    return _MESH


def _sc_deg(row4):
    f = pl.kernel(
        _deg_body,
        out_type=jax.ShapeDtypeStruct((NC, NP), jnp.float32),
        mesh=_mesh(),
        scratch_types=[
            pltpu.VMEM((NBATCH, B), jnp.int32),
            pltpu.VMEM((B,), jnp.float32),
            pltpu.VMEM((RPT,), jnp.float32),
            pltpu.SemaphoreType.DMA,
            pltpu.VMEM_SHARED((NP,), jnp.float32),
        ],
    )
    return f(row4)


def _sc_spmm(x, comb):
    f = pl.kernel(
        _spmm_body,
        out_type=jax.ShapeDtypeStruct((NC, NP, D), jnp.float32),
        mesh=_mesh(),
        scratch_types=[
            pltpu.VMEM((CH, 128), jnp.int32),
            pltpu.VMEM((CH, 128), jnp.int32),
            pltpu.VMEM((B, D), jnp.float32),
            pltpu.VMEM((B, D), jnp.float32),
            pltpu.VMEM((B, D), jnp.float32),
            pltpu.VMEM((B, D), jnp.float32),
            pltpu.VMEM((B, D), jnp.float32),
            pltpu.SemaphoreType.DMA,
            pltpu.SemaphoreType.DMA,
            pltpu.SemaphoreType.DMA,
            pltpu.SemaphoreType.DMA,
            pltpu.SemaphoreType.DMA,
            pltpu.SemaphoreType.DMA,
            pltpu.SemaphoreType.DMA,
            pltpu.SemaphoreType.DMA,
            pltpu.SemaphoreType.DMA,
            pltpu.SemaphoreType.DMA,
            pltpu.SemaphoreType.DMA,
            pltpu.VMEM_SHARED((NP, D), jnp.float32),
        ],
    )
    return f(x, comb)


_R = 2000


def _dis_body(pdeg_ref, dis_ref):
    deg = pdeg_ref[0] + pdeg_ref[1]
    dis_ref[...] = jnp.where(deg > 0, lax.rsqrt(deg), 0.0)


def _tc_dis(pdeg):
    p3 = pdeg.reshape(NC, NP // 128, 128)
    f = pl.pallas_call(
        _dis_body,
        out_shape=jax.ShapeDtypeStruct((NP // 128, 128), jnp.float32),
    )
    return f(p3)


def _prep_body(dis_ref, emb_ref, x_ref):
    i = pl.program_id(0)
    d = dis_ref[pl.ds(i * _R, _R), :]
    x_ref[...] = emb_ref[...] * d


def _tc_prep(dis_col, emb0):
    f = pl.pallas_call(
        _prep_body,
        grid=(N // _R,),
        in_specs=[
            pl.BlockSpec((N, 1), lambda i: (0, 0)),
            pl.BlockSpec((_R, D), lambda i: (i, 0)),
        ],
        out_specs=pl.BlockSpec((_R, D), lambda i: (i, 0)),
        out_shape=jax.ShapeDtypeStruct((N, D), jnp.float32),
    )
    return f(dis_col, emb0)


def _comb_body(p_ref, dis_ref, acc_ref, accout_ref, x_ref):
    i = pl.program_id(0)
    t = p_ref[0] + p_ref[1]
    d = dis_ref[pl.ds(i * _R, _R), :]
    e = d * t
    accout_ref[...] = acc_ref[...] + e
    x_ref[...] = d * e


def _final_body(p_ref, dis_ref, acc_ref, out_ref):
    i = pl.program_id(0)
    t = p_ref[0] + p_ref[1]
    d = dis_ref[pl.ds(i * _R, _R), :]
    out_ref[...] = 0.25 * (acc_ref[...] + d * t)


def _tc_combine(p, dis_col, acc, last):
    body = _final_body if last else _comb_body
    oshape = jax.ShapeDtypeStruct((N, D), jnp.float32)
    ospec = pl.BlockSpec((_R, D), lambda i: (i, 0))
    f = pl.pallas_call(
        body,
        grid=(N // _R,),
        in_specs=[
            pl.BlockSpec((NC, _R, D), lambda i: (0, i, 0)),
            pl.BlockSpec((N, 1), lambda i: (0, 0)),
            pl.BlockSpec((_R, D), lambda i: (i, 0)),
        ],
        out_specs=ospec if last else (ospec, ospec),
        out_shape=oshape if last else (oshape, oshape),
    )
    return f(p, dis_col, acc)


def kernel(edge_index, user_emb, item_emb):
    row = jnp.concatenate([edge_index[0], edge_index[1]])
    col = jnp.concatenate([edge_index[1], edge_index[0]])
    emb0 = jnp.concatenate([user_emb, item_emb], axis=0)
    row4 = row.reshape(NC, NS, NBATCH, B)
    col4 = col.reshape(NC, NS, NBATCH, B)
    pad = jnp.zeros((NC, NS, NBATCH, 64 - B), jnp.int32)
    comb = jnp.concatenate([col4, pad, row4, pad], axis=-1)

    pdeg = _sc_deg(row4)
    dis_flat = _tc_dis(pdeg)
    dis_col = dis_flat.reshape(-1)[:N][:, None]

    x = _tc_prep(dis_col, emb0)
    acc = emb0
    for l in range(NLAYERS):
        p = _sc_spmm(x, comb)
        if l < NLAYERS - 1:
            acc, x = _tc_combine(p, dis_col, acc, last=False)
        else:
            final = _tc_combine(p, dis_col, acc, last=True)
    return (final[:NU], final[NU:])
